# initial kernel scaffold (unmeasured)
import functools

import jax
import jax.numpy as jnp
from jax import lax
from jax.experimental import pallas as pl
from jax.experimental.pallas import tpu as pltpu

N_DEV = 4


def kernel(x, w_mat):
    m_loc, k = x.shape
    _, n_loc = w_mat.shape
    m_half = m_loc // 2
    m_glob = N_DEV * m_loc
    n_hops = N_DEV - 1

    x = x.astype(jnp.bfloat16)
    w_mat = w_mat.astype(jnp.bfloat16)

    def body(
        x_ref, w_ref, out_ref,
        comm_cw, comm_ccw,
        send_cw, recv_cw, send_ccw, recv_ccw,
        amax_box, amax_recv, amax_send_sems, amax_recv_sems,
    ):
        me = lax.axis_index("i")
        right = lax.rem(me + 1, N_DEV)
        left = lax.rem(me + N_DEV - 1, N_DEV)

        bar = pltpu.get_barrier_semaphore()
        for off in range(1, N_DEV):
            peer = lax.rem(me + off, N_DEV)
            pl.semaphore_signal(
                bar, inc=1, device_id=(peer,),
                device_id_type=pl.DeviceIdType.MESH,
            )
        pl.semaphore_wait(bar, N_DEV - 1)

        cw = []
        ccw = []
        for h in range(n_hops):
            src_cw = x_ref.at[pl.ds(0, m_half), :] if h == 0 else comm_cw.at[h - 1]
            src_ccw = (
                x_ref.at[pl.ds(m_half, m_half), :] if h == 0 else comm_ccw.at[h - 1]
            )
            cw.append(pltpu.make_async_remote_copy(
                src_ref=src_cw, dst_ref=comm_cw.at[h],
                send_sem=send_cw.at[h], recv_sem=recv_cw.at[h],
                device_id=(right,), device_id_type=pl.DeviceIdType.MESH,
            ))
            ccw.append(pltpu.make_async_remote_copy(
                src_ref=src_ccw, dst_ref=comm_ccw.at[h],
                send_sem=send_ccw.at[h], recv_sem=recv_ccw.at[h],
                device_id=(left,), device_id_type=pl.DeviceIdType.MESH,
            ))

        def gemm_relu(a):
            acc = jax.lax.dot_general(
                a, w_ref[...],
                dimension_numbers=(((1,), (0,)), ((), ())),
                preferred_element_type=jnp.float32,
            )
            return jnp.maximum(acc, 0.0)

        cw[0].start()
        ccw[0].start()
        own = gemm_relu(x_ref[...])
        out_ref[pl.ds(me * m_loc, m_loc), :] = own
        amaxes = [jnp.max(own)]

        for h in range(n_hops):
            cw[h].wait_recv()
            ccw[h].wait_recv()
            if h + 1 < n_hops:
                cw[h + 1].start()
                ccw[h + 1].start()
            o_cw = lax.rem(me + N_DEV - (h + 1), N_DEV)
            o_ccw = lax.rem(me + (h + 1), N_DEV)
            top = gemm_relu(comm_cw[h])
            out_ref[pl.ds(o_cw * m_loc, m_half), :] = top
            amaxes.append(jnp.max(top))
            bot = gemm_relu(comm_ccw[h])
            out_ref[pl.ds(o_ccw * m_loc + m_half, m_half), :] = bot
            amaxes.append(jnp.max(bot))

        for h in range(n_hops):
            cw[h].wait_send()
            ccw[h].wait_send()

        local_amax = functools.reduce(jnp.maximum, amaxes)
        amax_box[...] = jnp.full((8, 128), local_amax, jnp.float32)
        a_rdmas = []
        for off in range(1, N_DEV):
            peer = lax.rem(me + off, N_DEV)
            r = pltpu.make_async_remote_copy(
                src_ref=amax_box, dst_ref=amax_recv.at[off - 1],
                send_sem=amax_send_sems.at[off - 1],
                recv_sem=amax_recv_sems.at[off - 1],
                device_id=(peer,), device_id_type=pl.DeviceIdType.MESH,
            )
            r.start()
            a_rdmas.append(r)
        g = local_amax
        for r in a_rdmas:
            r.wait_recv()
        for j in range(N_DEV - 1):
            g = jnp.maximum(g, jnp.max(amax_recv[j]))

        scale = g / 448.0
        inv = 448.0 / g
        for blk in range(N_DEV):
            rows = pl.ds(blk * m_loc, m_loc)
            v = out_ref[rows, :]
            q = jnp.minimum(v * inv, 448.0).astype(jnp.float8_e4m3fn)
            out_ref[rows, :] = q.astype(jnp.float32) * scale

        for r in a_rdmas:
            r.wait_send()

    return pl.pallas_call(
        body,
        out_shape=jax.ShapeDtypeStruct((m_glob, n_loc), jnp.float32),
        in_specs=[
            pl.BlockSpec(memory_space=pltpu.VMEM),
            pl.BlockSpec(memory_space=pltpu.VMEM),
        ],
        out_specs=pl.BlockSpec(memory_space=pltpu.VMEM),
        scratch_shapes=[
            pltpu.VMEM((n_hops, m_half, k), jnp.bfloat16),
            pltpu.VMEM((n_hops, m_half, k), jnp.bfloat16),
            pltpu.SemaphoreType.DMA((n_hops,)),
            pltpu.SemaphoreType.DMA((n_hops,)),
            pltpu.SemaphoreType.DMA((n_hops,)),
            pltpu.SemaphoreType.DMA((n_hops,)),
            pltpu.VMEM((8, 128), jnp.float32),
            pltpu.VMEM((N_DEV - 1, 8, 128), jnp.float32),
            pltpu.SemaphoreType.DMA((N_DEV - 1,)),
            pltpu.SemaphoreType.DMA((N_DEV - 1,)),
        ],
        compiler_params=pltpu.CompilerParams(
            collective_id=0,
            vmem_limit_bytes=120 * 1024 * 1024,
        ),
    )(x, w_mat)


# baseline (device time: 270092 ns/iter reference)
import functools

import jax
import jax.numpy as jnp
from jax import lax
from jax.experimental import pallas as pl
from jax.experimental.pallas import tpu as pltpu

N_DEV = 4


def kernel(x, w_mat):
    m_loc, k = x.shape
    _, n_loc = w_mat.shape
    m_half = m_loc // 2
    m_glob = N_DEV * m_loc
    n_hops = N_DEV - 1
    n_blocks = 2 * N_DEV

    x = x.astype(jnp.bfloat16)
    w_mat = w_mat.astype(jnp.bfloat16)

    def body(
        x_ref, w_ref, out_ref,
        own_cw, own_ccw, comm_cw, comm_ccw, stage,
        send_cw, recv_cw, send_ccw, recv_ccw,
        stage_sems, store_sems, load_sems,
        amax_box, amax_recv, amax_send_sems, amax_recv_sems,
    ):
        me = lax.axis_index("i")
        right = lax.rem(me + 1, N_DEV)
        left = lax.rem(me + N_DEV - 1, N_DEV)

        bar = pltpu.get_barrier_semaphore()
        for off in range(1, N_DEV):
            peer = lax.rem(me + off, N_DEV)
            pl.semaphore_signal(
                bar, inc=1, device_id=(peer,),
                device_id_type=pl.DeviceIdType.MESH,
            )
        pl.semaphore_wait(bar, N_DEV - 1)

        st_top = pltpu.make_async_copy(
            x_ref.at[pl.ds(0, m_half), :], own_cw, stage_sems.at[0]
        )
        st_bot = pltpu.make_async_copy(
            x_ref.at[pl.ds(m_half, m_half), :], own_ccw, stage_sems.at[1]
        )
        st_top.start()
        st_bot.start()
        st_top.wait()
        st_bot.wait()

        cw = []
        ccw = []
        for h in range(n_hops):
            cw.append(pltpu.make_async_remote_copy(
                src_ref=own_cw if h == 0 else comm_cw.at[h - 1],
                dst_ref=comm_cw.at[h],
                send_sem=send_cw.at[h], recv_sem=recv_cw.at[h],
                device_id=(right,), device_id_type=pl.DeviceIdType.MESH,
            ))
            ccw.append(pltpu.make_async_remote_copy(
                src_ref=own_ccw if h == 0 else comm_ccw.at[h - 1],
                dst_ref=comm_ccw.at[h],
                send_sem=send_ccw.at[h], recv_sem=recv_ccw.at[h],
                device_id=(left,), device_id_type=pl.DeviceIdType.MESH,
            ))

        amaxes = []
        out_stores = []

        def do_block(chunk, row_start):
            b = len(out_stores)
            slot = b % 2
            if b >= 2:
                out_stores[b - 2].wait()
            acc = jax.lax.dot_general(
                chunk, w_ref[...],
                dimension_numbers=(((1,), (0,)), ((), ())),
                preferred_element_type=jnp.float32,
            )
            acc = jnp.maximum(acc, 0.0)
            stage[slot, :, :] = acc
            st = pltpu.make_async_copy(
                stage.at[slot],
                out_ref.at[pl.ds(row_start, m_half), :],
                store_sems.at[slot],
            )
            st.start()
            out_stores.append(st)
            amaxes.append(jnp.max(acc))

        cw[0].start()
        ccw[0].start()
        do_block(own_cw[:, :], me * m_loc)
        do_block(own_ccw[:, :], me * m_loc + m_half)

        for h in range(n_hops):
            cw[h].wait_recv()
            ccw[h].wait_recv()
            if h + 1 < n_hops:
                cw[h + 1].start()
                ccw[h + 1].start()
            o_cw = lax.rem(me + N_DEV - (h + 1), N_DEV)
            o_ccw = lax.rem(me + (h + 1), N_DEV)
            do_block(comm_cw[h], o_cw * m_loc)
            do_block(comm_ccw[h], o_ccw * m_loc + m_half)

        for h in range(n_hops):
            cw[h].wait_send()
            ccw[h].wait_send()

        local_amax = functools.reduce(jnp.maximum, amaxes)
        amax_box[...] = jnp.full((8, 128), local_amax, jnp.float32)
        a_rdmas = []
        for off in range(1, N_DEV):
            peer = lax.rem(me + off, N_DEV)
            r = pltpu.make_async_remote_copy(
                src_ref=amax_box, dst_ref=amax_recv.at[off - 1],
                send_sem=amax_send_sems.at[off - 1],
                recv_sem=amax_recv_sems.at[off - 1],
                device_id=(peer,), device_id_type=pl.DeviceIdType.MESH,
            )
            r.start()
            a_rdmas.append(r)

        out_stores[-2].wait()
        out_stores[-1].wait()
        g = local_amax
        for r in a_rdmas:
            r.wait_recv()
        for j in range(N_DEV - 1):
            g = jnp.maximum(g, jnp.max(amax_recv[j]))

        scale = g / 448.0
        inv = 448.0 / g

        def quant_dequant(v):
            vs = jnp.minimum(v * inv, 448.0)
            u = lax.bitcast_convert_type(vs, jnp.uint32)
            r = (u + 0x7FFFF + ((u >> 20) & 1)) & jnp.uint32(0xFFF00000)
            qn = lax.bitcast_convert_type(r, jnp.float32)
            qs = jnp.round(vs * 512.0) * (1.0 / 512.0)
            q = jnp.where(vs < 0.015625, qs, qn)
            return q * scale

        ep_stores = []
        for b in range(n_blocks):
            slot = b % 2
            rows = pl.ds(b * m_half, m_half)
            if b >= 2:
                ep_stores[b - 2].wait()
            ld = pltpu.make_async_copy(
                out_ref.at[rows, :], stage.at[slot], load_sems.at[slot]
            )
            ld.start()
            ld.wait()
            stage[slot, :, :] = quant_dequant(stage[slot, :, :])
            st = pltpu.make_async_copy(
                stage.at[slot], out_ref.at[rows, :], store_sems.at[slot]
            )
            st.start()
            ep_stores.append(st)
        ep_stores[-2].wait()
        ep_stores[-1].wait()

        for r in a_rdmas:
            r.wait_send()

    return pl.pallas_call(
        body,
        out_shape=jax.ShapeDtypeStruct((m_glob, n_loc), jnp.float32),
        in_specs=[
            pl.BlockSpec(memory_space=pl.ANY),
            pl.BlockSpec(memory_space=pltpu.VMEM),
        ],
        out_specs=pl.BlockSpec(memory_space=pl.ANY),
        scratch_shapes=[
            pltpu.VMEM((m_half, k), jnp.bfloat16),
            pltpu.VMEM((m_half, k), jnp.bfloat16),
            pltpu.VMEM((n_hops, m_half, k), jnp.bfloat16),
            pltpu.VMEM((n_hops, m_half, k), jnp.bfloat16),
            pltpu.VMEM((2, m_half, n_loc), jnp.float32),
            pltpu.SemaphoreType.DMA((n_hops,)),
            pltpu.SemaphoreType.DMA((n_hops,)),
            pltpu.SemaphoreType.DMA((n_hops,)),
            pltpu.SemaphoreType.DMA((n_hops,)),
            pltpu.SemaphoreType.DMA((2,)),
            pltpu.SemaphoreType.DMA((2,)),
            pltpu.SemaphoreType.DMA((2,)),
            pltpu.VMEM((8, 128), jnp.float32),
            pltpu.VMEM((N_DEV - 1, 8, 128), jnp.float32),
            pltpu.SemaphoreType.DMA((N_DEV - 1,)),
            pltpu.SemaphoreType.DMA((N_DEV - 1,)),
        ],
        compiler_params=pltpu.CompilerParams(
            collective_id=0,
            vmem_limit_bytes=100 * 1024 * 1024,
        ),
    )(x, w_mat)


# device time: 247940 ns/iter; 1.0893x vs baseline; 1.0893x over previous
import functools

import jax
import jax.numpy as jnp
from jax import lax
from jax.experimental import pallas as pl
from jax.experimental.pallas import tpu as pltpu

N_DEV = 4


def kernel(x, w_mat):
    m_loc, k = x.shape
    _, n_loc = w_mat.shape
    m_half = m_loc // 2
    m_glob = N_DEV * m_loc
    n_hops = N_DEV - 1
    n_blocks = 2 * N_DEV

    x = x.astype(jnp.bfloat16)
    w_mat = w_mat.astype(jnp.bfloat16)

    def body(
        x_ref, w_ref, out_ref,
        own_cw, own_ccw, comm_cw, comm_ccw, stage,
        send_cw, recv_cw, send_ccw, recv_ccw,
        stage_sems, store_sems, load_sems,
        amax_box, amax_recv, amax_send_sems, amax_recv_sems,
    ):
        me = lax.axis_index("i")
        right = lax.rem(me + 1, N_DEV)
        left = lax.rem(me + N_DEV - 1, N_DEV)

        bar = pltpu.get_barrier_semaphore()
        for off in range(1, N_DEV):
            peer = lax.rem(me + off, N_DEV)
            pl.semaphore_signal(
                bar, inc=1, device_id=(peer,),
                device_id_type=pl.DeviceIdType.MESH,
            )
        pl.semaphore_wait(bar, N_DEV - 1)

        st_top = pltpu.make_async_copy(
            x_ref.at[pl.ds(0, m_half), :], own_cw, stage_sems.at[0]
        )
        st_bot = pltpu.make_async_copy(
            x_ref.at[pl.ds(m_half, m_half), :], own_ccw, stage_sems.at[1]
        )
        st_top.start()
        st_bot.start()
        st_top.wait()
        st_bot.wait()

        cw = []
        ccw = []
        for h in range(n_hops):
            cw.append(pltpu.make_async_remote_copy(
                src_ref=own_cw if h == 0 else comm_cw.at[h - 1],
                dst_ref=comm_cw.at[h],
                send_sem=send_cw.at[h], recv_sem=recv_cw.at[h],
                device_id=(right,), device_id_type=pl.DeviceIdType.MESH,
            ))
            ccw.append(pltpu.make_async_remote_copy(
                src_ref=own_ccw if h == 0 else comm_ccw.at[h - 1],
                dst_ref=comm_ccw.at[h],
                send_sem=send_ccw.at[h], recv_sem=recv_ccw.at[h],
                device_id=(left,), device_id_type=pl.DeviceIdType.MESH,
            ))

        amaxes = []
        out_stores = []
        m_q = m_half // 2

        def do_block(chunk, row_start):
            b = len(out_stores)
            s0, s1 = 2 * (b % 2), 2 * (b % 2) + 1
            if b >= 2:
                out_stores[b - 2][0].wait()
                out_stores[b - 2][1].wait()
            acc = jax.lax.dot_general(
                chunk, w_ref[...],
                dimension_numbers=(((1,), (0,)), ((), ())),
                preferred_element_type=jnp.float32,
            )
            acc = jnp.maximum(acc, 0.0)
            stage[s0, :, :] = acc[:m_q, :]
            stage[s1, :, :] = acc[m_q:, :]
            sts = []
            for i, s in enumerate((s0, s1)):
                st = pltpu.make_async_copy(
                    stage.at[s],
                    out_ref.at[pl.ds(row_start + i * m_q, m_q), :],
                    store_sems.at[s],
                )
                st.start()
                sts.append(st)
            out_stores.append(tuple(sts))
            amaxes.append(jnp.max(acc))

        cw[0].start()
        ccw[0].start()
        do_block(own_cw[:, :], me * m_loc)
        do_block(own_ccw[:, :], me * m_loc + m_half)

        for h in range(n_hops):
            cw[h].wait_recv()
            ccw[h].wait_recv()
            if h + 1 < n_hops:
                cw[h + 1].start()
                ccw[h + 1].start()
            o_cw = lax.rem(me + N_DEV - (h + 1), N_DEV)
            o_ccw = lax.rem(me + (h + 1), N_DEV)
            do_block(comm_cw[h], o_cw * m_loc)
            do_block(comm_ccw[h], o_ccw * m_loc + m_half)

        for h in range(n_hops):
            cw[h].wait_send()
            ccw[h].wait_send()

        local_amax = functools.reduce(jnp.maximum, amaxes)
        amax_box[...] = jnp.full((8, 128), local_amax, jnp.float32)
        a_rdmas = []
        for off in range(1, N_DEV):
            peer = lax.rem(me + off, N_DEV)
            r = pltpu.make_async_remote_copy(
                src_ref=amax_box, dst_ref=amax_recv.at[off - 1],
                send_sem=amax_send_sems.at[off - 1],
                recv_sem=amax_recv_sems.at[off - 1],
                device_id=(peer,), device_id_type=pl.DeviceIdType.MESH,
            )
            r.start()
            a_rdmas.append(r)

        def make_load(j):
            return pltpu.make_async_copy(
                out_ref.at[pl.ds(j * m_q, m_q), :], stage.at[j % 4],
                load_sems.at[j % 4],
            )

        loads = {}
        out_stores[-2][0].wait()
        out_stores[-2][1].wait()
        for j in (0, 1):
            loads[j] = make_load(j)
            loads[j].start()
        out_stores[-1][0].wait()
        out_stores[-1][1].wait()
        for j in (2, 3):
            loads[j] = make_load(j)
            loads[j].start()

        g = local_amax
        for r in a_rdmas:
            r.wait_recv()
        for j in range(N_DEV - 1):
            g = jnp.maximum(g, jnp.max(amax_recv[j]))

        scale = g / 448.0
        inv = 448.0 / g

        def quant_dequant(v):
            vs = jnp.minimum(v * inv, 448.0)
            u = lax.bitcast_convert_type(vs, jnp.uint32)
            r = (u + 0x7FFFF + ((u >> 20) & 1)) & jnp.uint32(0xFFF00000)
            qn = lax.bitcast_convert_type(r, jnp.float32)
            qs = jnp.round(vs * 512.0) * (1.0 / 512.0)
            q = jnp.where(vs < 0.015625, qs, qn)
            return q * scale

        n_sub = 2 * n_blocks
        ep_stores = []
        for j in range(n_sub):
            slot = j % 4
            rows = pl.ds(j * m_q, m_q)
            if j >= 2:
                ep_stores[j - 2].wait()
                if j + 2 < n_sub:
                    loads[j + 2] = make_load(j + 2)
                    loads[j + 2].start()
            loads[j].wait()
            stage[slot, :, :] = quant_dequant(stage[slot, :, :])
            st = pltpu.make_async_copy(
                stage.at[slot], out_ref.at[rows, :], store_sems.at[slot]
            )
            st.start()
            ep_stores.append(st)
        ep_stores[-2].wait()
        ep_stores[-1].wait()

        for r in a_rdmas:
            r.wait_send()

    return pl.pallas_call(
        body,
        out_shape=jax.ShapeDtypeStruct((m_glob, n_loc), jnp.float32),
        in_specs=[
            pl.BlockSpec(memory_space=pl.ANY),
            pl.BlockSpec(memory_space=pltpu.VMEM),
        ],
        out_specs=pl.BlockSpec(memory_space=pl.ANY),
        scratch_shapes=[
            pltpu.VMEM((m_half, k), jnp.bfloat16),
            pltpu.VMEM((m_half, k), jnp.bfloat16),
            pltpu.VMEM((n_hops, m_half, k), jnp.bfloat16),
            pltpu.VMEM((n_hops, m_half, k), jnp.bfloat16),
            pltpu.VMEM((4, m_half // 2, n_loc), jnp.float32),
            pltpu.SemaphoreType.DMA((n_hops,)),
            pltpu.SemaphoreType.DMA((n_hops,)),
            pltpu.SemaphoreType.DMA((n_hops,)),
            pltpu.SemaphoreType.DMA((n_hops,)),
            pltpu.SemaphoreType.DMA((2,)),
            pltpu.SemaphoreType.DMA((4,)),
            pltpu.SemaphoreType.DMA((4,)),
            pltpu.VMEM((8, 128), jnp.float32),
            pltpu.VMEM((N_DEV - 1, 8, 128), jnp.float32),
            pltpu.SemaphoreType.DMA((N_DEV - 1,)),
            pltpu.SemaphoreType.DMA((N_DEV - 1,)),
        ],
        compiler_params=pltpu.CompilerParams(
            collective_id=0,
            vmem_limit_bytes=100 * 1024 * 1024,
        ),
    )(x, w_mat)


# device time: 236723 ns/iter; 1.1410x vs baseline; 1.0474x over previous
import functools

import jax
import jax.numpy as jnp
from jax import lax
from jax.experimental import pallas as pl
from jax.experimental.pallas import tpu as pltpu

N_DEV = 4


def kernel(x, w_mat):
    m_loc, k = x.shape
    _, n_loc = w_mat.shape
    m_half = m_loc // 2
    m_q = m_half // 2
    m_glob = N_DEV * m_loc
    n_hops = N_DEV - 1
    n_sub = 2 * N_DEV * 2
    n_wp = k // m_q

    x = x.astype(jnp.bfloat16)
    w_mat = w_mat.astype(jnp.bfloat16)

    def body(
        x_ref, w_ref, out_ref,
        own_cw, own_ccw, comm_cw, comm_ccw, stage,
        send_cw, recv_cw, send_ccw, recv_ccw,
        stage_sems, store_sems, load_sems,
        amax_box, amax_recv, amax_send_sems, amax_recv_sems,
    ):
        me = lax.axis_index("i")
        right = lax.rem(me + 1, N_DEV)
        left = lax.rem(me + N_DEV - 1, N_DEV)

        bar = pltpu.get_barrier_semaphore()
        for off in range(1, N_DEV):
            peer = lax.rem(me + off, N_DEV)
            pl.semaphore_signal(
                bar, inc=1, device_id=(peer,),
                device_id_type=pl.DeviceIdType.MESH,
            )
        pl.semaphore_wait(bar, N_DEV - 1)

        def rdma(src, dst, sems, h, dev):
            return pltpu.make_async_remote_copy(
                src_ref=src, dst_ref=dst,
                send_sem=sems[0].at[h], recv_sem=sems[1].at[h],
                device_id=(dev,), device_id_type=pl.DeviceIdType.MESH,
            )

        cw_sems = (send_cw, recv_cw)
        ccw_sems = (send_ccw, recv_ccw)
        st_top = pltpu.make_async_copy(
            x_ref.at[pl.ds(0, m_half), :], own_cw, stage_sems.at[0]
        )
        st_bot = pltpu.make_async_copy(
            x_ref.at[pl.ds(m_half, m_half), :], own_ccw, stage_sems.at[1]
        )
        st_top.start()
        st_bot.start()
        st_top.wait()
        st_bot.wait()

        cw = [
            rdma(own_cw, comm_cw.at[0], cw_sems, 0, right),
            rdma(comm_cw.at[0], comm_cw.at[1], cw_sems, 1, right),
            rdma(comm_cw.at[1, pl.ds(0, m_q), :],
                 comm_cw.at[2, pl.ds(0, m_q), :], cw_sems, 2, right),
            rdma(comm_cw.at[1, pl.ds(m_q, m_q), :],
                 comm_cw.at[2, pl.ds(m_q, m_q), :], cw_sems, 3, right),
        ]
        ccw = [
            rdma(own_ccw, comm_ccw.at[0], ccw_sems, 0, left),
            rdma(comm_ccw.at[0], comm_ccw.at[1], ccw_sems, 1, left),
            rdma(comm_ccw.at[1, pl.ds(0, m_q), :],
                 comm_ccw.at[2, pl.ds(0, m_q), :], ccw_sems, 2, left),
            rdma(comm_ccw.at[1, pl.ds(m_q, m_q), :],
                 comm_ccw.at[2, pl.ds(m_q, m_q), :], ccw_sems, 3, left),
        ]
        cw[0].start()
        ccw[0].start()

        amaxes = []
        subs = []

        def do_sub(chunk, row_start):
            i = len(subs)
            slot = i % 4
            if i >= 4:
                subs[i - 4].wait()
            acc = jax.lax.dot_general(
                chunk, w_ref[...],
                dimension_numbers=(((1,), (0,)), ((), ())),
                preferred_element_type=jnp.float32,
            )
            acc = jnp.maximum(acc, 0.0)
            stage[slot, :, :] = acc
            st = pltpu.make_async_copy(
                stage.at[slot],
                out_ref.at[pl.ds(row_start, m_q), :],
                store_sems.at[slot],
            )
            st.start()
            subs.append(st)
            amaxes.append(jnp.max(acc))

        def do_block(chunk, row_start):
            i = len(subs)
            s0, s1 = i % 4, (i + 1) % 4
            if i >= 4:
                subs[i - 4].wait()
            if i + 1 >= 4:
                subs[i - 3].wait()
            acc = jax.lax.dot_general(
                chunk, w_ref[...],
                dimension_numbers=(((1,), (0,)), ((), ())),
                preferred_element_type=jnp.float32,
            )
            acc = jnp.maximum(acc, 0.0)
            stage[s0, :, :] = acc[:m_q, :]
            stage[s1, :, :] = acc[m_q:, :]
            for j, s in enumerate((s0, s1)):
                st = pltpu.make_async_copy(
                    stage.at[s],
                    out_ref.at[pl.ds(row_start + j * m_q, m_q), :],
                    store_sems.at[s],
                )
                st.start()
                subs.append(st)
            amaxes.append(jnp.max(acc))

        do_block(own_cw[:, :], me * m_loc)
        do_block(own_ccw[:, :], me * m_loc + m_half)

        cw[0].wait_recv()
        ccw[0].wait_recv()
        cw[1].start()
        ccw[1].start()
        o_cw = lax.rem(me + N_DEV - 1, N_DEV)
        o_ccw = lax.rem(me + 1, N_DEV)
        do_block(comm_cw[0], o_cw * m_loc)
        do_block(comm_ccw[0], o_ccw * m_loc + m_half)

        cw[1].wait_recv()
        ccw[1].wait_recv()
        for r in (cw[2], cw[3], ccw[2], ccw[3]):
            r.start()
        o_cw = lax.rem(me + N_DEV - 2, N_DEV)
        o_ccw = lax.rem(me + 2, N_DEV)
        do_block(comm_cw[1], o_cw * m_loc)
        do_block(comm_ccw[1], o_ccw * m_loc + m_half)

        o_cw = lax.rem(me + N_DEV - 3, N_DEV)
        o_ccw = lax.rem(me + 3, N_DEV)
        cw[2].wait_recv()
        do_sub(comm_cw[2, pl.ds(0, m_q), :], o_cw * m_loc)
        ccw[2].wait_recv()
        do_sub(comm_ccw[2, pl.ds(0, m_q), :], o_ccw * m_loc + m_half)
        cw[3].wait_recv()
        do_sub(comm_cw[2, pl.ds(m_q, m_q), :], o_cw * m_loc + m_q)
        ccw[3].wait_recv()
        do_sub(comm_ccw[2, pl.ds(m_q, m_q), :], o_ccw * m_loc + m_half + m_q)

        for r in cw + ccw:
            r.wait_send()

        local_amax = functools.reduce(jnp.maximum, amaxes)
        amax_box[...] = jnp.full((8, 128), local_amax, jnp.float32)
        a_rdmas = []
        for off in range(1, N_DEV):
            peer = lax.rem(me + off, N_DEV)
            r = pltpu.make_async_remote_copy(
                src_ref=amax_box, dst_ref=amax_recv.at[off - 1],
                send_sem=amax_send_sems.at[off - 1],
                recv_sem=amax_recv_sems.at[off - 1],
                device_id=(peer,), device_id_type=pl.DeviceIdType.MESH,
            )
            r.start()
            a_rdmas.append(r)

        def make_load(j):
            return pltpu.make_async_copy(
                out_ref.at[pl.ds(j * m_q, m_q), :], stage.at[j % 4],
                load_sems.at[j % 4],
            )

        loads = {}
        for j in range(4):
            subs[n_sub - 4 + j].wait()
            loads[j] = make_load(j)
            loads[j].start()

        g = local_amax
        for r in a_rdmas:
            r.wait_recv()
        for j in range(N_DEV - 1):
            g = jnp.maximum(g, jnp.max(amax_recv[j]))

        scale = g / 448.0
        inv = 448.0 / g

        def quant_dequant(v):
            vs = jnp.minimum(v * inv, 448.0)
            u = lax.bitcast_convert_type(vs, jnp.uint32)
            r = (u + 0x7FFFF + ((u >> 20) & 1)) & jnp.uint32(0xFFF00000)
            q = lax.bitcast_convert_type(r, jnp.float32)
            return q * scale

        ep_stores = []
        for j in range(n_sub):
            slot = j % 4
            rows = pl.ds(j * m_q, m_q)
            if j >= 2:
                ep_stores[j - 2].wait()
                if j + 2 < n_sub:
                    loads[j + 2] = make_load(j + 2)
                    loads[j + 2].start()
            loads[j].wait()
            stage[slot, :, :] = quant_dequant(stage[slot, :, :])
            st = pltpu.make_async_copy(
                stage.at[slot], out_ref.at[rows, :], store_sems.at[slot]
            )
            st.start()
            ep_stores.append(st)
        ep_stores[-2].wait()
        ep_stores[-1].wait()

        for r in a_rdmas:
            r.wait_send()

    return pl.pallas_call(
        body,
        out_shape=jax.ShapeDtypeStruct((m_glob, n_loc), jnp.float32),
        in_specs=[
            pl.BlockSpec(memory_space=pl.ANY),
            pl.BlockSpec(memory_space=pltpu.VMEM),
        ],
        out_specs=pl.BlockSpec(memory_space=pl.ANY),
        scratch_shapes=[
            pltpu.VMEM((m_half, k), jnp.bfloat16),
            pltpu.VMEM((m_half, k), jnp.bfloat16),
            pltpu.VMEM((n_hops, m_half, k), jnp.bfloat16),
            pltpu.VMEM((n_hops, m_half, k), jnp.bfloat16),
            pltpu.VMEM((4, m_q, n_loc), jnp.float32),
            pltpu.SemaphoreType.DMA((4,)),
            pltpu.SemaphoreType.DMA((4,)),
            pltpu.SemaphoreType.DMA((4,)),
            pltpu.SemaphoreType.DMA((4,)),
            pltpu.SemaphoreType.DMA((2,)),
            pltpu.SemaphoreType.DMA((4,)),
            pltpu.SemaphoreType.DMA((4,)),
            pltpu.VMEM((8, 128), jnp.float32),
            pltpu.VMEM((N_DEV - 1, 8, 128), jnp.float32),
            pltpu.SemaphoreType.DMA((N_DEV - 1,)),
            pltpu.SemaphoreType.DMA((N_DEV - 1,)),
        ],
        compiler_params=pltpu.CompilerParams(
            collective_id=0,
            vmem_limit_bytes=100 * 1024 * 1024,
        ),
    )(x, w_mat)
